# Initial kernel scaffold; baseline (speedup 1.0000x reference)
import jax
import jax.numpy as jnp
from jax import lax
from jax.experimental import pallas as pl
from jax.experimental.pallas import tpu as pltpu
from jax.experimental.pallas import tpu_sc as plsc

N = 10000
NP = 10240
DIN = 128
HID = 128
LAT = 64
E = 320000
NC, NS = 2, 16
NW = NC * NS
CH = 128
KCH = 80
EP = NW * KCH * CH
PAD = NP - N
RPT = NP // NS

_MESH = plsc.VectorSubcoreMesh(
    core_axis_name="c", subcore_axis_name="s", num_cores=NC, num_subcores=NS)


def _deg_body(dstp, zeros_r, ones_r, out, idx_v, zb, ones_v, acc):
    c = lax.axis_index("c")
    s = lax.axis_index("s")
    wid = c * NS + s
    pltpu.sync_copy(zeros_r, zb)
    pltpu.sync_copy(ones_r, ones_v)
    pltpu.sync_copy(zb, acc.at[pl.ds(s * RPT, RPT)])
    plsc.subcore_barrier()
    pltpu.sync_copy(dstp.at[wid], idx_v)

    def body(j, carry):
        pltpu.sync_copy(ones_v, acc.at[idx_v.at[j]], add=True)
        return carry

    lax.fori_loop(0, KCH, body, 0)
    plsc.subcore_barrier()
    pltpu.sync_copy(acc.at[pl.ds(s * RPT, RPT)], zb)
    pltpu.sync_copy(zb, out.at[c, pl.ds(s * RPT, RPT)])


_deg_call = pl.kernel(
    _deg_body,
    out_type=jax.ShapeDtypeStruct((NC, NP), jnp.float32),
    mesh=_MESH,
    scratch_types=[
        pltpu.VMEM((KCH, CH), jnp.int32),
        pltpu.VMEM((RPT,), jnp.float32),
        pltpu.VMEM((CH,), jnp.float32),
        pltpu.VMEM_SHARED((NP,), jnp.float32),
    ],
)


def _agg_body(y, srcp, dstp, zrows, out, src_v, dst_v, rows, acc, sem):
    c = lax.axis_index("c")
    s = lax.axis_index("s")
    wid = c * NS + s
    pltpu.sync_copy(zrows, rows)
    for k in range(RPT // CH):
        pltpu.sync_copy(rows, acc.at[pl.ds(s * RPT + k * CH, CH)])
    plsc.subcore_barrier()
    pltpu.sync_copy(srcp.at[wid], src_v)
    pltpu.sync_copy(dstp.at[wid], dst_v)

    def body(j, carry):
        pltpu.async_copy(y.at[src_v.at[j]], rows, sem).wait()
        pltpu.sync_copy(rows, acc.at[dst_v.at[j]], add=True)
        return carry

    lax.fori_loop(0, KCH, body, 0)
    plsc.subcore_barrier()
    for k in range(RPT // CH):
        off = s * RPT + k * CH
        pltpu.sync_copy(acc.at[pl.ds(off, CH)], rows)
        pltpu.sync_copy(rows, out.at[c, pl.ds(off, CH)])


def _make_agg_call(d):
    return pl.kernel(
        _agg_body,
        out_type=jax.ShapeDtypeStruct((NC, NP, d), jnp.float32),
        mesh=_MESH,
        scratch_types=[
            pltpu.VMEM((KCH, CH), jnp.int32),
            pltpu.VMEM((KCH, CH), jnp.int32),
            pltpu.VMEM((CH, d), jnp.float32),
            pltpu.VMEM_SHARED((NP, d), jnp.float32),
            pltpu.SemaphoreType.DMA,
        ],
    )


_agg_call_hid = _make_agg_call(HID)
_agg_call_lat = _make_agg_call(LAT)


_BR = 1024


def _mm_scale_body(x_ref, w_ref, degp_ref, y_ref, dinv_ref):
    deg = degp_ref[0, :] + degp_ref[1, :] + 1.0
    dinv = lax.rsqrt(deg)
    y = jnp.dot(x_ref[...], w_ref[...], preferred_element_type=jnp.float32)
    y_ref[...] = y * dinv[:, None]
    dinv_ref[...] = dinv


_mm_scale_call = pl.pallas_call(
    _mm_scale_body,
    grid=(NP // _BR,),
    in_specs=[
        pl.BlockSpec((_BR, DIN), lambda i: (i, 0)),
        pl.BlockSpec((DIN, HID), lambda i: (0, 0)),
        pl.BlockSpec((NC, _BR), lambda i: (0, i)),
    ],
    out_specs=[
        pl.BlockSpec((_BR, HID), lambda i: (i, 0)),
        pl.BlockSpec((_BR,), lambda i: (i,)),
    ],
    out_shape=[
        jax.ShapeDtypeStruct((NP, HID), jnp.float32),
        jax.ShapeDtypeStruct((NP,), jnp.float32),
    ],
)


def _fuse1_body(agg_ref, y1_ref, dinv_ref, b1_ref, w2_ref, y2_ref):
    a = agg_ref[0] + agg_ref[1] + y1_ref[...]
    dinv = dinv_ref[...]
    z = jnp.maximum(a * dinv[:, None] + b1_ref[...][None, :], 0.0)
    y2 = jnp.dot(z, w2_ref[...], preferred_element_type=jnp.float32)
    y2_ref[...] = y2 * dinv[:, None]


_fuse1_call = pl.pallas_call(
    _fuse1_body,
    grid=(NP // _BR,),
    in_specs=[
        pl.BlockSpec((NC, _BR, HID), lambda i: (0, i, 0)),
        pl.BlockSpec((_BR, HID), lambda i: (i, 0)),
        pl.BlockSpec((_BR,), lambda i: (i,)),
        pl.BlockSpec((HID,), lambda i: (0,)),
        pl.BlockSpec((HID, LAT), lambda i: (0, 0)),
    ],
    out_specs=pl.BlockSpec((_BR, LAT), lambda i: (i, 0)),
    out_shape=jax.ShapeDtypeStruct((NP, LAT), jnp.float32),
)


def _fuse2_body(agg_ref, y2_ref, dinv_ref, b2_ref, wmu_ref, bmu_ref,
                wlv_ref, blv_ref, mu_ref, lv_ref):
    a = agg_ref[0] + agg_ref[1] + y2_ref[...]
    z = a * dinv_ref[...][:, None] + b2_ref[...][None, :]
    mu_ref[...] = (jnp.dot(z, wmu_ref[...], preferred_element_type=jnp.float32)
                   + bmu_ref[...][None, :])
    lv_ref[...] = (jnp.dot(z, wlv_ref[...], preferred_element_type=jnp.float32)
                   + blv_ref[...][None, :])


_fuse2_call = pl.pallas_call(
    _fuse2_body,
    grid=(NP // _BR,),
    in_specs=[
        pl.BlockSpec((NC, _BR, LAT), lambda i: (0, i, 0)),
        pl.BlockSpec((_BR, LAT), lambda i: (i, 0)),
        pl.BlockSpec((_BR,), lambda i: (i,)),
        pl.BlockSpec((LAT,), lambda i: (0,)),
        pl.BlockSpec((LAT, LAT), lambda i: (0, 0)),
        pl.BlockSpec((LAT,), lambda i: (0,)),
        pl.BlockSpec((LAT, LAT), lambda i: (0, 0)),
        pl.BlockSpec((LAT,), lambda i: (0,)),
    ],
    out_specs=[
        pl.BlockSpec((_BR, LAT), lambda i: (i, 0)),
        pl.BlockSpec((_BR, LAT), lambda i: (i, 0)),
    ],
    out_shape=[
        jax.ShapeDtypeStruct((NP, LAT), jnp.float32),
        jax.ShapeDtypeStruct((NP, LAT), jnp.float32),
    ],
)


def kernel(x, edge_index, W1, b1, W2, b2, W_mu, b_mu, W_lv, b_lv):
    xp = jnp.pad(x, ((0, NP - N), (0, 0)))
    pad_idx = (jnp.arange(EP - E, dtype=jnp.int32) % PAD) + N
    srcp = jnp.concatenate([edge_index[0], pad_idx]).reshape(NW, KCH, CH)
    dstp = jnp.concatenate([edge_index[1], pad_idx]).reshape(NW, KCH, CH)
    zeros_r = jnp.zeros((RPT,), jnp.float32)
    ones_r = jnp.ones((CH,), jnp.float32)
    zrows_h = jnp.zeros((CH, HID), jnp.float32)
    zrows_l = jnp.zeros((CH, LAT), jnp.float32)

    degp = _deg_call(dstp, zeros_r, ones_r)
    y1, dinv = _mm_scale_call(xp, W1, degp)
    agg1 = _agg_call_hid(y1, srcp, dstp, zrows_h)
    y2 = _fuse1_call(agg1, y1, dinv, b1, W2)
    agg2 = _agg_call_lat(y2, srcp, dstp, zrows_l)
    mu_p, lv_p = _fuse2_call(agg2, y2, dinv, b2, W_mu, b_mu, W_lv, b_lv)
    return (mu_p[:N], lv_p[:N])


# speedup vs baseline: 25.3892x; 25.3892x over previous
import jax
import jax.numpy as jnp
from jax import lax
from jax.experimental import pallas as pl
from jax.experimental.pallas import tpu as pltpu
from jax.experimental.pallas import tpu_sc as plsc

N = 10000
NP = 10240
DIN = 128
HID = 128
LAT = 64
E = 320000
NC, NS = 2, 16
NW = NC * NS
CH = 128
KCH = 80
EP = NW * KCH * CH
PAD = NP - N
RPT = NP // NS

_MESH = plsc.VectorSubcoreMesh(
    core_axis_name="c", subcore_axis_name="s", num_cores=NC, num_subcores=NS)


def _deg_body(dstp, zeros_r, ones_r, out, idx_v, zb, ones_v, acc):
    c = lax.axis_index("c")
    s = lax.axis_index("s")
    wid = c * NS + s
    pltpu.sync_copy(zeros_r, zb)
    pltpu.sync_copy(ones_r, ones_v)
    pltpu.sync_copy(zb, acc.at[pl.ds(s * RPT, RPT)])
    plsc.subcore_barrier()
    pltpu.sync_copy(dstp.at[wid], idx_v)

    def body(j, carry):
        pltpu.sync_copy(ones_v, acc.at[idx_v.at[j]], add=True)
        return carry

    lax.fori_loop(0, KCH, body, 0)
    plsc.subcore_barrier()
    pltpu.sync_copy(acc.at[pl.ds(s * RPT, RPT)], zb)
    pltpu.sync_copy(zb, out.at[c, pl.ds(s * RPT, RPT)])


_deg_call = pl.kernel(
    _deg_body,
    out_type=jax.ShapeDtypeStruct((NC, NP), jnp.float32),
    mesh=_MESH,
    scratch_types=[
        pltpu.VMEM((KCH, CH), jnp.int32),
        pltpu.VMEM((RPT,), jnp.float32),
        pltpu.VMEM((CH,), jnp.float32),
        pltpu.VMEM_SHARED((NP,), jnp.float32),
    ],
)


def _agg_body(y, srcp, dstp, zrows, out, src_v, dst_v, rows, acc, sem):
    c = lax.axis_index("c")
    s = lax.axis_index("s")
    wid = c * NS + s
    pltpu.sync_copy(zrows, rows)
    for k in range(RPT // CH):
        pltpu.sync_copy(rows, acc.at[pl.ds(s * RPT + k * CH, CH)])
    plsc.subcore_barrier()
    pltpu.sync_copy(srcp.at[wid], src_v)
    pltpu.sync_copy(dstp.at[wid], dst_v)

    def body(j, carry):
        pltpu.async_copy(y.at[src_v.at[j]], rows, sem).wait()
        pltpu.sync_copy(rows, acc.at[dst_v.at[j]], add=True)
        return carry

    lax.fori_loop(0, KCH, body, 0)
    plsc.subcore_barrier()
    for k in range(RPT // CH):
        off = s * RPT + k * CH
        pltpu.sync_copy(acc.at[pl.ds(off, CH)], rows)
        pltpu.sync_copy(rows, out.at[c, pl.ds(off, CH)])


def _make_agg_call(d):
    return pl.kernel(
        _agg_body,
        out_type=jax.ShapeDtypeStruct((NC, NP, d), jnp.float32),
        mesh=_MESH,
        compiler_params=pltpu.CompilerParams(use_tc_tiling_on_sc=(d == HID)),
        scratch_types=[
            pltpu.VMEM((KCH, CH), jnp.int32),
            pltpu.VMEM((KCH, CH), jnp.int32),
            pltpu.VMEM((CH, d), jnp.float32),
            pltpu.VMEM_SHARED((NP, d), jnp.float32),
            pltpu.SemaphoreType.DMA,
        ],
    )


_agg_call_hid = _make_agg_call(HID)
_agg_call_lat = _make_agg_call(LAT)


_BR = 1024


def _mm_scale_body(x_ref, w_ref, degp_ref, y_ref, dinv_ref):
    deg = degp_ref[0, :] + degp_ref[1, :] + 1.0
    dinv = lax.rsqrt(deg)
    y = jnp.dot(x_ref[...], w_ref[...], preferred_element_type=jnp.float32)
    y_ref[...] = y * dinv[:, None]
    dinv_ref[...] = dinv


_mm_scale_call = pl.pallas_call(
    _mm_scale_body,
    grid=(NP // _BR,),
    in_specs=[
        pl.BlockSpec((_BR, DIN), lambda i: (i, 0)),
        pl.BlockSpec((DIN, HID), lambda i: (0, 0)),
        pl.BlockSpec((NC, _BR), lambda i: (0, i)),
    ],
    out_specs=[
        pl.BlockSpec((_BR, HID), lambda i: (i, 0)),
        pl.BlockSpec((_BR,), lambda i: (i,)),
    ],
    out_shape=[
        jax.ShapeDtypeStruct((NP, HID), jnp.float32),
        jax.ShapeDtypeStruct((NP,), jnp.float32),
    ],
)


def _fuse1_body(agg_ref, y1_ref, dinv_ref, b1_ref, w2_ref, y2_ref):
    a = agg_ref[0] + agg_ref[1] + y1_ref[...]
    dinv = dinv_ref[...]
    z = jnp.maximum(a * dinv[:, None] + b1_ref[...][None, :], 0.0)
    y2 = jnp.dot(z, w2_ref[...], preferred_element_type=jnp.float32)
    y2_ref[...] = y2 * dinv[:, None]


_fuse1_call = pl.pallas_call(
    _fuse1_body,
    grid=(NP // _BR,),
    in_specs=[
        pl.BlockSpec((NC, _BR, HID), lambda i: (0, i, 0)),
        pl.BlockSpec((_BR, HID), lambda i: (i, 0)),
        pl.BlockSpec((_BR,), lambda i: (i,)),
        pl.BlockSpec((HID,), lambda i: (0,)),
        pl.BlockSpec((HID, LAT), lambda i: (0, 0)),
    ],
    out_specs=pl.BlockSpec((_BR, LAT), lambda i: (i, 0)),
    out_shape=jax.ShapeDtypeStruct((NP, LAT), jnp.float32),
)


def _fuse2_body(agg_ref, y2_ref, dinv_ref, b2_ref, wmu_ref, bmu_ref,
                wlv_ref, blv_ref, mu_ref, lv_ref):
    a = agg_ref[0] + agg_ref[1] + y2_ref[...]
    z = a * dinv_ref[...][:, None] + b2_ref[...][None, :]
    mu_ref[...] = (jnp.dot(z, wmu_ref[...], preferred_element_type=jnp.float32)
                   + bmu_ref[...][None, :])
    lv_ref[...] = (jnp.dot(z, wlv_ref[...], preferred_element_type=jnp.float32)
                   + blv_ref[...][None, :])


_fuse2_call = pl.pallas_call(
    _fuse2_body,
    grid=(NP // _BR,),
    in_specs=[
        pl.BlockSpec((NC, _BR, LAT), lambda i: (0, i, 0)),
        pl.BlockSpec((_BR, LAT), lambda i: (i, 0)),
        pl.BlockSpec((_BR,), lambda i: (i,)),
        pl.BlockSpec((LAT,), lambda i: (0,)),
        pl.BlockSpec((LAT, LAT), lambda i: (0, 0)),
        pl.BlockSpec((LAT,), lambda i: (0,)),
        pl.BlockSpec((LAT, LAT), lambda i: (0, 0)),
        pl.BlockSpec((LAT,), lambda i: (0,)),
    ],
    out_specs=[
        pl.BlockSpec((_BR, LAT), lambda i: (i, 0)),
        pl.BlockSpec((_BR, LAT), lambda i: (i, 0)),
    ],
    out_shape=[
        jax.ShapeDtypeStruct((NP, LAT), jnp.float32),
        jax.ShapeDtypeStruct((NP, LAT), jnp.float32),
    ],
)


def kernel(x, edge_index, W1, b1, W2, b2, W_mu, b_mu, W_lv, b_lv):
    xp = jnp.pad(x, ((0, NP - N), (0, 0)))
    pad_idx = (jnp.arange(EP - E, dtype=jnp.int32) % PAD) + N
    srcp = jnp.concatenate([edge_index[0], pad_idx]).reshape(NW, KCH, CH)
    dstp = jnp.concatenate([edge_index[1], pad_idx]).reshape(NW, KCH, CH)
    zeros_r = jnp.zeros((RPT,), jnp.float32)
    ones_r = jnp.ones((CH,), jnp.float32)
    zrows_h = jnp.zeros((CH, HID), jnp.float32)
    zrows_l = jnp.zeros((CH, LAT), jnp.float32)

    degp = _deg_call(dstp, zeros_r, ones_r)
    y1, dinv = _mm_scale_call(xp, W1, degp)
    agg1 = _agg_call_hid(y1, srcp, dstp, zrows_h)
    y2 = _fuse1_call(agg1, y1, dinv, b1, W2)
    agg2 = _agg_call_lat(y2, srcp, dstp, zrows_l)
    mu_p, lv_p = _fuse2_call(agg2, y2, dinv, b2, W_mu, b_mu, W_lv, b_lv)
    return (mu_p[:N], lv_p[:N])


# speedup vs baseline: 31.8432x; 1.2542x over previous
import jax
import jax.numpy as jnp
from jax import lax
from jax.experimental import pallas as pl
from jax.experimental.pallas import tpu as pltpu
from jax.experimental.pallas import tpu_sc as plsc

N = 10000
NP = 10240
DIN = 128
HID = 128
LAT = 64
E = 320000
NC, NS = 2, 16
NW = NC * NS
CH = 128
KCH = 80
KCH1 = 160
EP = NW * KCH * CH
PAD = NP - N
RPT = NP // NS

_MESH = plsc.VectorSubcoreMesh(
    core_axis_name="c", subcore_axis_name="s", num_cores=NC, num_subcores=NS)

_SC_LINEAR = pltpu.CompilerParams(use_tc_tiling_on_sc=False)


def _deg_body(dstp, zeros_r, ones_r, out, idx_v, zb, ones_v, acc):
    c = lax.axis_index("c")
    s = lax.axis_index("s")
    wid = c * NS + s
    pltpu.sync_copy(zeros_r, zb)
    pltpu.sync_copy(ones_r, ones_v)
    pltpu.sync_copy(zb, acc.at[pl.ds(s * RPT, RPT)])
    plsc.subcore_barrier()
    pltpu.sync_copy(dstp.at[wid], idx_v)

    def body(j, carry):
        pltpu.sync_copy(ones_v, acc.at[idx_v.at[j]], add=True)
        return carry

    lax.fori_loop(0, KCH, body, 0)
    plsc.subcore_barrier()
    pltpu.sync_copy(acc.at[pl.ds(s * RPT, RPT)], zb)
    pltpu.sync_copy(zb, out.at[c, pl.ds(s * RPT, RPT)])


_deg_call = pl.kernel(
    _deg_body,
    out_type=jax.ShapeDtypeStruct((NC, NP), jnp.float32),
    mesh=_MESH,
    scratch_types=[
        pltpu.VMEM((KCH, CH), jnp.int32),
        pltpu.VMEM((RPT,), jnp.float32),
        pltpu.VMEM((CH,), jnp.float32),
        pltpu.VMEM_SHARED((NP,), jnp.float32),
    ],
)


def _gather_scatter_loop(y, src_v, dst_v, rows_a, rows_b, acc, sem_a, sem_b,
                         kch):
    pltpu.async_copy(y.at[src_v.at[0]], rows_a, sem_a)

    def body(p, carry):
        j = 2 * p
        pltpu.async_copy(y.at[src_v.at[j + 1]], rows_b, sem_b)
        pltpu.make_async_copy(y.at[src_v.at[j]], rows_a, sem_a).wait()
        pltpu.sync_copy(rows_a, acc.at[dst_v.at[j]], add=True)
        jn = jnp.minimum(j + 2, kch - 1)
        pltpu.async_copy(y.at[src_v.at[jn]], rows_a, sem_a)
        pltpu.make_async_copy(y.at[src_v.at[j + 1]], rows_b, sem_b).wait()
        pltpu.sync_copy(rows_b, acc.at[dst_v.at[j + 1]], add=True)
        return carry

    lax.fori_loop(0, kch // 2, body, 0)
    pltpu.make_async_copy(y.at[src_v.at[kch - 1]], rows_a, sem_a).wait()


def _zero_acc_slice(zrows, rows_a, acc, s):
    pltpu.sync_copy(zrows, rows_a)
    for k in range(RPT // CH):
        pltpu.sync_copy(rows_a, acc.at[pl.ds(s * RPT + k * CH, CH)])


def _writeback(acc, rows_a, out, c, s):
    for k in range(RPT // CH):
        off = s * RPT + k * CH
        pltpu.sync_copy(acc.at[pl.ds(off, CH)], rows_a)
        pltpu.sync_copy(rows_a, out.at[c, pl.ds(off, CH)])


def _agg_cs_body(ycs, srcp, dstp, zrows, out, src_v, dst_v, rows_a, rows_b,
                 acc, sem_a, sem_b):
    c = lax.axis_index("c")
    s = lax.axis_index("s")
    _zero_acc_slice(zrows, rows_a, acc, s)
    plsc.subcore_barrier()
    pltpu.sync_copy(srcp.at[s], src_v)
    pltpu.sync_copy(dstp.at[s], dst_v)
    _gather_scatter_loop(ycs.at[c], src_v, dst_v, rows_a, rows_b, acc,
                         sem_a, sem_b, KCH1)
    plsc.subcore_barrier()
    _writeback(acc, rows_a, out, c, s)


_agg_cs_call = pl.kernel(
    _agg_cs_body,
    out_type=jax.ShapeDtypeStruct((NC, NP, LAT), jnp.float32),
    mesh=_MESH,
    compiler_params=_SC_LINEAR,
    scratch_types=[
        pltpu.VMEM((KCH1, CH), jnp.int32),
        pltpu.VMEM((KCH1, CH), jnp.int32),
        pltpu.VMEM((CH, LAT), jnp.float32),
        pltpu.VMEM((CH, LAT), jnp.float32),
        pltpu.VMEM_SHARED((NP, LAT), jnp.float32),
        pltpu.SemaphoreType.DMA,
        pltpu.SemaphoreType.DMA,
    ],
)


def _agg_es_body(y, srcp, dstp, zrows, out, src_v, dst_v, rows_a, rows_b,
                 acc, sem_a, sem_b):
    c = lax.axis_index("c")
    s = lax.axis_index("s")
    wid = c * NS + s
    _zero_acc_slice(zrows, rows_a, acc, s)
    plsc.subcore_barrier()
    pltpu.sync_copy(srcp.at[wid], src_v)
    pltpu.sync_copy(dstp.at[wid], dst_v)
    _gather_scatter_loop(y, src_v, dst_v, rows_a, rows_b, acc,
                         sem_a, sem_b, KCH)
    plsc.subcore_barrier()
    _writeback(acc, rows_a, out, c, s)


_agg_es_call = pl.kernel(
    _agg_es_body,
    out_type=jax.ShapeDtypeStruct((NC, NP, LAT), jnp.float32),
    mesh=_MESH,
    compiler_params=_SC_LINEAR,
    scratch_types=[
        pltpu.VMEM((KCH, CH), jnp.int32),
        pltpu.VMEM((KCH, CH), jnp.int32),
        pltpu.VMEM((CH, LAT), jnp.float32),
        pltpu.VMEM((CH, LAT), jnp.float32),
        pltpu.VMEM_SHARED((NP, LAT), jnp.float32),
        pltpu.SemaphoreType.DMA,
        pltpu.SemaphoreType.DMA,
    ],
)


_BR = 1024


def _mm_scale_body(x_ref, w_ref, degp_ref, y_ref, dinv_ref):
    deg = degp_ref[0, :] + degp_ref[1, :] + 1.0
    dinv = lax.rsqrt(deg)
    y = jnp.dot(x_ref[...], w_ref[...], preferred_element_type=jnp.float32)
    y = y * dinv[:, None]
    y_ref[0] = y[:, :LAT]
    y_ref[1] = y[:, LAT:]
    dinv_ref[...] = dinv


_mm_scale_call = pl.pallas_call(
    _mm_scale_body,
    grid=(NP // _BR,),
    in_specs=[
        pl.BlockSpec((_BR, DIN), lambda i: (i, 0)),
        pl.BlockSpec((DIN, HID), lambda i: (0, 0)),
        pl.BlockSpec((NC, _BR), lambda i: (0, i)),
    ],
    out_specs=[
        pl.BlockSpec((NC, _BR, LAT), lambda i: (0, i, 0)),
        pl.BlockSpec((_BR,), lambda i: (i,)),
    ],
    out_shape=[
        jax.ShapeDtypeStruct((NC, NP, LAT), jnp.float32),
        jax.ShapeDtypeStruct((NP,), jnp.float32),
    ],
)


def _fuse1_body(agg_ref, y1_ref, dinv_ref, b1_ref, w2_ref, y2_ref):
    dinv = dinv_ref[...]
    a = jnp.concatenate([agg_ref[0] + y1_ref[0], agg_ref[1] + y1_ref[1]],
                        axis=1)
    z = jnp.maximum(a * dinv[:, None] + b1_ref[...][None, :], 0.0)
    y2 = jnp.dot(z, w2_ref[...], preferred_element_type=jnp.float32)
    y2_ref[...] = y2 * dinv[:, None]


_fuse1_call = pl.pallas_call(
    _fuse1_body,
    grid=(NP // _BR,),
    in_specs=[
        pl.BlockSpec((NC, _BR, LAT), lambda i: (0, i, 0)),
        pl.BlockSpec((NC, _BR, LAT), lambda i: (0, i, 0)),
        pl.BlockSpec((_BR,), lambda i: (i,)),
        pl.BlockSpec((HID,), lambda i: (0,)),
        pl.BlockSpec((HID, LAT), lambda i: (0, 0)),
    ],
    out_specs=pl.BlockSpec((_BR, LAT), lambda i: (i, 0)),
    out_shape=jax.ShapeDtypeStruct((NP, LAT), jnp.float32),
)


def _fuse2_body(agg_ref, y2_ref, dinv_ref, b2_ref, wmu_ref, bmu_ref,
                wlv_ref, blv_ref, mu_ref, lv_ref):
    a = agg_ref[0] + agg_ref[1] + y2_ref[...]
    z = a * dinv_ref[...][:, None] + b2_ref[...][None, :]
    mu_ref[...] = (jnp.dot(z, wmu_ref[...], preferred_element_type=jnp.float32)
                   + bmu_ref[...][None, :])
    lv_ref[...] = (jnp.dot(z, wlv_ref[...], preferred_element_type=jnp.float32)
                   + blv_ref[...][None, :])


_fuse2_call = pl.pallas_call(
    _fuse2_body,
    grid=(NP // _BR,),
    in_specs=[
        pl.BlockSpec((NC, _BR, LAT), lambda i: (0, i, 0)),
        pl.BlockSpec((_BR, LAT), lambda i: (i, 0)),
        pl.BlockSpec((_BR,), lambda i: (i,)),
        pl.BlockSpec((LAT,), lambda i: (0,)),
        pl.BlockSpec((LAT, LAT), lambda i: (0, 0)),
        pl.BlockSpec((LAT,), lambda i: (0,)),
        pl.BlockSpec((LAT, LAT), lambda i: (0, 0)),
        pl.BlockSpec((LAT,), lambda i: (0,)),
    ],
    out_specs=[
        pl.BlockSpec((_BR, LAT), lambda i: (i, 0)),
        pl.BlockSpec((_BR, LAT), lambda i: (i, 0)),
    ],
    out_shape=[
        jax.ShapeDtypeStruct((NP, LAT), jnp.float32),
        jax.ShapeDtypeStruct((NP, LAT), jnp.float32),
    ],
)


def kernel(x, edge_index, W1, b1, W2, b2, W_mu, b_mu, W_lv, b_lv):
    xp = jnp.pad(x, ((0, NP - N), (0, 0)))
    pad_idx = (jnp.arange(EP - E, dtype=jnp.int32) % PAD) + N
    src = jnp.concatenate([edge_index[0], pad_idx])
    dst = jnp.concatenate([edge_index[1], pad_idx])
    srcp16 = src.reshape(NS, KCH1, CH)
    dstp16 = dst.reshape(NS, KCH1, CH)
    srcp32 = src.reshape(NW, KCH, CH)
    dstp32 = dst.reshape(NW, KCH, CH)
    zeros_r = jnp.zeros((RPT,), jnp.float32)
    ones_r = jnp.ones((CH,), jnp.float32)
    zrows = jnp.zeros((CH, LAT), jnp.float32)

    degp = _deg_call(dstp32, zeros_r, ones_r)
    y1cs, dinv = _mm_scale_call(xp, W1, degp)
    agg1 = _agg_cs_call(y1cs, srcp16, dstp16, zrows)
    y2 = _fuse1_call(agg1, y1cs, dinv, b1, W2)
    agg2 = _agg_es_call(y2, srcp32, dstp32, zrows)
    mu_p, lv_p = _fuse2_call(agg2, y2, dinv, b2, W_mu, b_mu, W_lv, b_lv)
    return (mu_p[:N], lv_p[:N])
